# Initial kernel scaffold; baseline (speedup 1.0000x reference)
import jax
import jax.numpy as jnp
from jax.experimental import pallas as pl


def kernel(x, edge_index, W1, a_src1, a_dst1, b1, W2, a_src2, a_dst2, b2, W3, a_src3, a_dst3, b3):
    raise NotImplementedError("write your pallas kernel here")


# speedup vs baseline: 1.0001x; 1.0001x over previous
import jax
import jax.numpy as jnp
from jax.experimental import pallas as pl
from jax.experimental.pallas import tpu as pltpu

N = 10000
H = 4


def _gat(x, src, dst, W, att_src, att_dst, bias, heads, out_ch, concat):
    h = (x @ W).reshape(N, heads, out_ch)
    a_src = jnp.sum(h * att_src, axis=-1)
    a_dst = jnp.sum(h * att_dst, axis=-1)
    alpha = a_src[src] + a_dst[dst]
    alpha = jax.nn.leaky_relu(alpha, negative_slope=0.2)
    amax = jax.ops.segment_max(alpha, dst, num_segments=N)
    amax = jnp.where(jnp.isfinite(amax), amax, 0.0)
    ex = jnp.exp(alpha - amax[dst])
    denom = jax.ops.segment_sum(ex, dst, num_segments=N)
    coef = ex / (denom[dst] + 1e-16)
    msg = h[src] * coef[:, :, None]
    out = jax.ops.segment_sum(msg, dst, num_segments=N)
    if concat:
        out = out.reshape(N, heads * out_ch)
    else:
        out = jnp.mean(out, axis=1)
    return out + bias


def _tail_kernel(h_ref, emb_ref, pred_ref):
    h = h_ref[...]
    emb_ref[...] = h
    m = jnp.max(h, axis=1, keepdims=True)
    lse = jnp.log(jnp.sum(jnp.exp(h - m), axis=1, keepdims=True)) + m
    pred_ref[...] = h - lse


def kernel(x, edge_index, W1, a_src1, a_dst1, b1, W2, a_src2, a_dst2, b2, W3, a_src3, a_dst3, b3):
    src = edge_index[0].astype(jnp.int32)
    dst = edge_index[1].astype(jnp.int32)
    loop = jnp.arange(N, dtype=jnp.int32)
    src = jnp.concatenate([src, loop])
    dst = jnp.concatenate([dst, loop])
    h = _gat(x, src, dst, W1, a_src1, a_dst1, b1, H, 256, True)
    h = jax.nn.relu(h)
    h = _gat(h, src, dst, W2, a_src2, a_dst2, b2, H, 256, True)
    h = jax.nn.relu(h)
    h = _gat(h, src, dst, W3, a_src3, a_dst3, b3, 1, 128, False)
    emb, pred = pl.pallas_call(
        _tail_kernel,
        out_shape=(
            jax.ShapeDtypeStruct((N, 128), jnp.float32),
            jax.ShapeDtypeStruct((N, 128), jnp.float32),
        ),
        grid=(10,),
        in_specs=[pl.BlockSpec((1000, 128), lambda i: (i, 0))],
        out_specs=(
            pl.BlockSpec((1000, 128), lambda i: (i, 0)),
            pl.BlockSpec((1000, 128), lambda i: (i, 0)),
        ),
    )(h)
    return (emb, pred)


# speedup vs baseline: 11.1615x; 11.1608x over previous
import functools

import jax
import jax.numpy as jnp
from jax import lax
from jax.experimental import pallas as pl
from jax.experimental.pallas import tpu as pltpu
from jax.experimental.pallas import tpu_sc as plsc

N = 10000
E_RAW = 160000
E_REAL = E_RAW + N
NC = 2
NS = 16
NW = NC * NS
EP = 172032
N2 = 10240
BA = 256
BC = 128
BN = 1000
NEG_SLOPE = 0.2

_SC_MESH = dict(core_axis_name="c", subcore_axis_name="s", num_cores=NC,
                num_subcores=NS)


def _mm_body(x_ref, w_ref, as_ref, ad_ref, h_ref, asd_ref, sv_ref, *,
             heads, oc, nch, chunked_in, b_ref=None):
    i = pl.program_id(0)
    if chunked_in:
        xs = [jnp.maximum(x_ref[c] + b_ref[c][None, :], 0.0) for c in range(8)]
        x = jnp.concatenate(xs, axis=1)
    else:
        x = x_ref[...]
    h = jnp.dot(x, w_ref[...], preferred_element_type=jnp.float32,
                precision=lax.Precision.HIGHEST)
    cw = h.shape[1] // nch
    for c in range(nch):
        h_ref[c] = h[:, c * cw:(c + 1) * cw]

    rows = []
    maxs = []
    for hd in range(heads):
        hh = h[:, hd * oc:(hd + 1) * oc]
        a_s = jnp.dot(hh, as_ref[hd], preferred_element_type=jnp.float32,
                      precision=lax.Precision.HIGHEST)
        rows.append(a_s)
        maxs.append(jnp.max(a_s))
    for hd in range(heads, 8):
        rows.append(jnp.zeros((BN,), jnp.float32))
        maxs.append(jnp.float32(0.0))
    for hd in range(heads):
        hh = h[:, hd * oc:(hd + 1) * oc]
        a_d = jnp.dot(hh, ad_ref[hd], preferred_element_type=jnp.float32,
                      precision=lax.Precision.HIGHEST)
        rows.append(a_d)
        maxs.append(jnp.max(a_d))
    for hd in range(heads, 8):
        rows.append(jnp.zeros((BN,), jnp.float32))
        maxs.append(jnp.float32(0.0))
    asd_ref[...] = jnp.stack(rows, axis=0)[None]

    mv = jnp.stack([jnp.full((128,), m, jnp.float32) for m in maxs[:8]],
                   axis=0)
    mv2 = jnp.stack([jnp.full((128,), m, jnp.float32) for m in maxs[8:]],
                    axis=0)
    cur = jnp.concatenate([mv[:4], mv2[:4]], axis=0)

    @pl.when(i == 0)
    def _():
        sv_ref[...] = jnp.full((8, 128), -1e30, jnp.float32)

    sv_ref[...] = jnp.maximum(sv_ref[...], cur)


def _mm_call(x, w, att_s, att_d, *, heads, oc, nch, chunked_in, bias=None):
    din = w.shape[0]
    dout = w.shape[1]
    grid = (N // BN,)
    if chunked_in:
        in_specs = [
            pl.BlockSpec((8, BN, 128), lambda i: (0, i, 0)),
            pl.BlockSpec((8, 128), lambda i: (0, 0)),
            pl.BlockSpec((din, dout), lambda i: (0, 0)),
            pl.BlockSpec((8, oc), lambda i: (0, 0)),
            pl.BlockSpec((8, oc), lambda i: (0, 0)),
        ]
        args = (x, bias, w, att_s, att_d)

        def body(x_ref, b_ref, w_ref, as_ref, ad_ref, h_ref, asd_ref, sv_ref):
            _mm_body(x_ref, w_ref, as_ref, ad_ref, h_ref, asd_ref, sv_ref,
                     heads=heads, oc=oc, nch=nch, chunked_in=True, b_ref=b_ref)
    else:
        in_specs = [
            pl.BlockSpec((BN, din), lambda i: (i, 0)),
            pl.BlockSpec((din, dout), lambda i: (0, 0)),
            pl.BlockSpec((8, oc), lambda i: (0, 0)),
            pl.BlockSpec((8, oc), lambda i: (0, 0)),
        ]
        args = (x, w, att_s, att_d)

        def body(x_ref, w_ref, as_ref, ad_ref, h_ref, asd_ref, sv_ref):
            _mm_body(x_ref, w_ref, as_ref, ad_ref, h_ref, asd_ref, sv_ref,
                     heads=heads, oc=oc, nch=nch, chunked_in=False)

    cw = dout // nch
    h_resh, asd, sv = pl.pallas_call(
        body,
        grid=grid,
        in_specs=in_specs,
        out_specs=(
            pl.BlockSpec((nch, BN, cw), lambda i: (0, i, 0)),
            pl.BlockSpec((1, 16, BN), lambda i: (i, 0, 0)),
            pl.BlockSpec((8, 128), lambda i: (0, 0)),
        ),
        out_shape=(
            jax.ShapeDtypeStruct((nch, N, cw), jnp.float32),
            jax.ShapeDtypeStruct((N // BN, 16, BN), jnp.float32),
            jax.ShapeDtypeStruct((8, 128), jnp.float32),
        ),
    )(*args)
    asd = asd.transpose(1, 0, 2).reshape(16, N)
    return h_resh, asd, sv


def _pass_a_call(srcp, dstp, asdf, svf, heads):
    epw = EP // NW
    nb = epw // BA
    mesh = plsc.VectorSubcoreMesh(**_SC_MESH)

    @functools.partial(
        pl.kernel,
        out_type=(
            jax.ShapeDtypeStruct((heads * EP,), jnp.float32),
            jax.ShapeDtypeStruct((NW * heads * N2,), jnp.float32),
        ),
        mesh=mesh,
        compiler_params=pltpu.CompilerParams(needs_layout_passes=False),
        scratch_types=[
            pltpu.VMEM((N2,), jnp.float32),
            pltpu.VMEM((N2,), jnp.float32),
            pltpu.VMEM((N2,), jnp.float32),
            pltpu.VMEM((BA,), jnp.int32),
            pltpu.VMEM((BA,), jnp.int32),
            pltpu.VMEM((BA,), jnp.float32),
            pltpu.VMEM((16,), jnp.float32),
            pltpu.VMEM((16,), jnp.float32),
        ],
    )
    def k(src_hbm, dst_hbm, asd_hbm, sv_hbm, ex_hbm, part_hbm,
          s_tbl, d_tbl, denom, srcb, dstb, exb, svs, svd):
        wid = lax.axis_index("s") * NC + lax.axis_index("c")
        base = wid * epw
        for hd in range(heads):
            pltpu.sync_copy(asd_hbm.at[pl.ds(hd * N2, N2)], s_tbl)
            pltpu.sync_copy(asd_hbm.at[pl.ds((8 + hd) * N2, N2)], d_tbl)
            pltpu.sync_copy(sv_hbm.at[pl.ds(hd * 128, 16)], svs)
            pltpu.sync_copy(sv_hbm.at[pl.ds((4 + hd) * 128, 16)], svd)
            shift = svs[pl.ds(0, 16)][0] + svd[pl.ds(0, 16)][0]

            def zb(i, _):
                denom[pl.ds(i * 16, 16)] = jnp.zeros((16,), jnp.float32)
                return 0
            lax.fori_loop(0, N2 // 16, zb, 0)

            def batch(bi, _):
                off = base + bi * BA
                pltpu.sync_copy(src_hbm.at[pl.ds(off, BA)], srcb)
                pltpu.sync_copy(dst_hbm.at[pl.ds(off, BA)], dstb)

                def inner(j, _):
                    sl = pl.ds(j * 16, 16)
                    sv_idx = srcb[sl]
                    dv_idx = dstb[sl]
                    av = plsc.load_gather(s_tbl, [sv_idx])
                    bv = plsc.load_gather(d_tbl, [dv_idx])
                    al = av + bv
                    al = jnp.where(al > 0, al, NEG_SLOPE * al)
                    ex = jnp.exp(al - shift)
                    gid = off + j * 16 + lax.iota(jnp.int32, 16)
                    ex = jnp.where(gid < E_REAL, ex, 0.0)
                    exb[sl] = ex
                    plsc.addupdate_scatter(denom, [dv_idx], ex)
                    return 0
                lax.fori_loop(0, BA // 16, inner, 0)
                pltpu.sync_copy(exb, ex_hbm.at[pl.ds(hd * EP + off, BA)])
                return 0
            lax.fori_loop(0, nb, batch, 0)
            pltpu.sync_copy(
                denom, part_hbm.at[pl.ds(wid * heads * N2 + hd * N2, N2)])

    return k(srcp, dstp, asdf, svf)


def _reduce_call(partials, heads):
    m = heads * N2
    p2 = partials.reshape(NW, m)

    def body(p_ref, o_ref):
        s = jnp.sum(p_ref[...], axis=0)
        o_ref[...] = jnp.broadcast_to((1.0 / s)[None, :], (8, m))

    invd = pl.pallas_call(
        body,
        in_specs=[pl.BlockSpec((NW, m), lambda: (0, 0))],
        out_specs=pl.BlockSpec((8, m), lambda: (0, 0)),
        out_shape=jax.ShapeDtypeStruct((8, m), jnp.float32),
    )(p2)
    return invd.reshape(-1)


def _pass_c_call(srcp, dstp, exf, invdf, tbl, *, heads, ncht, cw):
    nch = ncht // NC
    etw = EP // NS
    nb = etw // BC
    zr = 64
    mesh = plsc.VectorSubcoreMesh(**_SC_MESH)

    @functools.partial(
        pl.kernel,
        out_type=jax.ShapeDtypeStruct((ncht * N, cw), jnp.float32),
        mesh=mesh,
        compiler_params=pltpu.CompilerParams(needs_layout_passes=False),
        scratch_types=[
            pltpu.VMEM_SHARED((N2, cw), jnp.float32),
            pltpu.VMEM((N2,), jnp.float32),
            pltpu.VMEM((BC,), jnp.int32),
            pltpu.VMEM((BC,), jnp.int32),
            pltpu.VMEM((BC,), jnp.float32),
            pltpu.VMEM((BC,), jnp.float32),
            pltpu.VMEM((BC, cw), jnp.float32),
            pltpu.VMEM((zr, cw), jnp.float32),
            pltpu.SemaphoreType.DMA,
        ],
    )
    def k(src_hbm, dst_hbm, ex_hbm, invd_hbm, tbl_hbm, out_hbm,
          acc, invd_tbl, srcb, dstb, exb, cfb, rowsb, zbuf, sem):
        cid = lax.axis_index("c")
        sid = lax.axis_index("s")
        ebase = sid * etw

        def zz(r, _):
            for c2 in range(cw // 16):
                zbuf[r, pl.ds(c2 * 16, 16)] = jnp.zeros((16,), jnp.float32)
            return 0
        lax.fori_loop(0, zr, zz, 0)

        for kk in range(nch):
            q = cid * nch + kk
            hd = q // (ncht // heads) if heads > 1 else q * 0
            for z in range(640 // zr):
                pltpu.sync_copy(
                    zbuf, acc.at[pl.ds(sid * 640 + z * zr, zr)])
            plsc.subcore_barrier()
            pltpu.sync_copy(
                invd_hbm.at[pl.ds(hd * N2, N2)], invd_tbl)

            def batch(bi, _):
                eoff = ebase + bi * BC
                pltpu.sync_copy(src_hbm.at[pl.ds(eoff, BC)], srcb)
                pltpu.sync_copy(dst_hbm.at[pl.ds(eoff, BC)], dstb)
                pltpu.sync_copy(ex_hbm.at[pl.ds(hd * EP + eoff, BC)], exb)

                def adj(j, _):
                    sl = pl.ds(j * 16, 16)
                    srcb[sl] = srcb[sl] + q * N
                    inv = plsc.load_gather(invd_tbl, [dstb[sl]])
                    cfb[sl] = exb[sl] * inv
                    return 0
                lax.fori_loop(0, BC // 16, adj, 0)

                pltpu.async_copy(tbl_hbm.at[srcb], rowsb, sem).wait()

                def wloop(g, _):
                    cf16 = cfb[pl.ds(g * 16, 16)]
                    for l in range(16):
                        cf = cf16[l]
                        r = g * 16 + l
                        for c2 in range(cw // 16):
                            sl2 = pl.ds(c2 * 16, 16)
                            rowsb[r, sl2] = rowsb[r, sl2] * cf
                    return 0
                lax.fori_loop(0, BC // 16, wloop, 0)

                pltpu.sync_copy(rowsb, acc.at[dstb], add=True)
                return 0
            lax.fori_loop(0, nb, batch, 0)
            plsc.subcore_barrier()

            @pl.when(sid < 15)
            def _():
                pltpu.sync_copy(
                    acc.at[pl.ds(sid * 640, 640)],
                    out_hbm.at[pl.ds(q * N + sid * 640, 640)])

            @pl.when(sid == 15)
            def _():
                pltpu.sync_copy(
                    acc.at[pl.ds(9600, 400)],
                    out_hbm.at[pl.ds(q * N + 9600, 400)])

    return k(srcp, dstp, exf, invdf, tbl)


def _tail_call(o3, b3):
    o3r = o3
    b3r = jnp.concatenate(
        [b3.reshape(1, 128), jnp.zeros((7, 128), jnp.float32)], axis=0)

    def body(o_ref, b_ref, emb_ref, pred_ref):
        h = o_ref[...] + b_ref[0][None, :]
        emb_ref[...] = h
        m = jnp.max(h, axis=1, keepdims=True)
        lse = jnp.log(jnp.sum(jnp.exp(h - m), axis=1, keepdims=True)) + m
        pred_ref[...] = h - lse

    emb, pred = pl.pallas_call(
        body,
        grid=(N // BN,),
        in_specs=[
            pl.BlockSpec((BN, 128), lambda i: (i, 0)),
            pl.BlockSpec((8, 128), lambda i: (0, 0)),
        ],
        out_specs=(
            pl.BlockSpec((BN, 128), lambda i: (i, 0)),
            pl.BlockSpec((BN, 128), lambda i: (i, 0)),
        ),
        out_shape=(
            jax.ShapeDtypeStruct((N, 128), jnp.float32),
            jax.ShapeDtypeStruct((N, 128), jnp.float32),
        ),
    )(o3r, b3r)
    return emb, pred


def _pad_att(a, oc):
    a2 = a.reshape(-1, oc)
    return jnp.concatenate(
        [a2, jnp.zeros((8 - a2.shape[0], oc), jnp.float32)], axis=0)


def _gat_layer(x, srcp, dstp, w, att_s, att_d, *, heads, oc, nch,
               agg_ncht, chunked_in, bias=None):
    h_resh, asd, sv = _mm_call(
        x, w, _pad_att(att_s, oc), _pad_att(att_d, oc),
        heads=heads, oc=oc, nch=nch, chunked_in=chunked_in, bias=bias)
    asdf = jnp.concatenate(
        [asd, jnp.zeros((16, N2 - N), jnp.float32)], axis=1).reshape(-1)
    exf, partials = _pass_a_call(srcp, dstp, asdf, sv.reshape(-1), heads)
    invdf = _reduce_call(partials, heads)
    tbl = h_resh.reshape(nch * N, 128)
    if agg_ncht != nch:
        tbl = jnp.concatenate([tbl] * (agg_ncht // nch), axis=0)
    out = _pass_c_call(
        srcp, dstp, exf, invdf, tbl, heads=heads, ncht=agg_ncht, cw=128)
    return out


def kernel(x, edge_index, W1, a_src1, a_dst1, b1, W2, a_src2, a_dst2, b2,
           W3, a_src3, a_dst3, b3):
    loop = jnp.arange(N, dtype=jnp.int32)
    pad = jnp.zeros((EP - E_REAL,), jnp.int32)
    srcp = jnp.concatenate([edge_index[0].astype(jnp.int32), loop, pad])
    dstp = jnp.concatenate([edge_index[1].astype(jnp.int32), loop, pad])

    o1 = _gat_layer(x, srcp, dstp, W1, a_src1, a_dst1,
                    heads=4, oc=256, nch=8, agg_ncht=8, chunked_in=False)
    o2 = _gat_layer(o1.reshape(8, N, 128), srcp, dstp, W2, a_src2, a_dst2,
                    heads=4, oc=256, nch=8, agg_ncht=8, chunked_in=True,
                    bias=b1.reshape(8, 128))
    o3 = _gat_layer(o2.reshape(8, N, 128), srcp, dstp, W3, a_src3, a_dst3,
                    heads=1, oc=128, nch=1, agg_ncht=2, chunked_in=True,
                    bias=b2.reshape(8, 128))
    return _tail_call(o3.reshape(2, N, 128)[0], b3)


# speedup vs baseline: 14.7651x; 1.3229x over previous
import functools

import jax
import jax.numpy as jnp
from jax import lax
from jax.experimental import pallas as pl
from jax.experimental.pallas import tpu as pltpu
from jax.experimental.pallas import tpu_sc as plsc

N = 10000
E_RAW = 160000
E_REAL = E_RAW + N
NC = 2
NS = 16
NW = NC * NS
EP = 172032
N2 = 10240
BA = 256
BC = 128
BN = 1000
NEG_SLOPE = 0.2

_SC_MESH = dict(core_axis_name="c", subcore_axis_name="s", num_cores=NC,
                num_subcores=NS)


def _mm_body(x_ref, w_ref, as_ref, ad_ref, h_ref, asd_ref, sv_ref, *,
             heads, oc, nch, chunked_in, b_ref=None):
    i = pl.program_id(0)
    if chunked_in:
        xs = [jnp.maximum(x_ref[c] + b_ref[c][None, :], 0.0) for c in range(8)]
        x = jnp.concatenate(xs, axis=1)
    else:
        x = x_ref[...]
    h = jnp.dot(x, w_ref[...], preferred_element_type=jnp.float32,
                precision=lax.Precision.HIGHEST)
    cw = h.shape[1] // nch
    for c in range(nch):
        h_ref[c] = h[:, c * cw:(c + 1) * cw]

    rows = []
    maxs = []
    for hd in range(heads):
        hh = h[:, hd * oc:(hd + 1) * oc]
        a_s = jnp.dot(hh, as_ref[hd], preferred_element_type=jnp.float32,
                      precision=lax.Precision.HIGHEST)
        rows.append(a_s)
        maxs.append(jnp.max(a_s))
    for hd in range(heads, 8):
        rows.append(jnp.zeros((BN,), jnp.float32))
        maxs.append(jnp.float32(0.0))
    for hd in range(heads):
        hh = h[:, hd * oc:(hd + 1) * oc]
        a_d = jnp.dot(hh, ad_ref[hd], preferred_element_type=jnp.float32,
                      precision=lax.Precision.HIGHEST)
        rows.append(a_d)
        maxs.append(jnp.max(a_d))
    for hd in range(heads, 8):
        rows.append(jnp.zeros((BN,), jnp.float32))
        maxs.append(jnp.float32(0.0))
    asd_ref[...] = jnp.stack(rows, axis=0)[None]

    mv = jnp.stack([jnp.full((128,), m, jnp.float32) for m in maxs[:8]],
                   axis=0)
    mv2 = jnp.stack([jnp.full((128,), m, jnp.float32) for m in maxs[8:]],
                    axis=0)
    cur = jnp.concatenate([mv[:4], mv2[:4]], axis=0)

    @pl.when(i == 0)
    def _():
        sv_ref[...] = jnp.full((8, 128), -1e30, jnp.float32)

    sv_ref[...] = jnp.maximum(sv_ref[...], cur)


def _mm_call(x, w, att_s, att_d, *, heads, oc, nch, chunked_in, bias=None):
    din = w.shape[0]
    dout = w.shape[1]
    grid = (N // BN,)
    if chunked_in:
        in_specs = [
            pl.BlockSpec((8, BN, 128), lambda i: (0, i, 0)),
            pl.BlockSpec((8, 128), lambda i: (0, 0)),
            pl.BlockSpec((din, dout), lambda i: (0, 0)),
            pl.BlockSpec((8, oc), lambda i: (0, 0)),
            pl.BlockSpec((8, oc), lambda i: (0, 0)),
        ]
        args = (x, bias, w, att_s, att_d)

        def body(x_ref, b_ref, w_ref, as_ref, ad_ref, h_ref, asd_ref, sv_ref):
            _mm_body(x_ref, w_ref, as_ref, ad_ref, h_ref, asd_ref, sv_ref,
                     heads=heads, oc=oc, nch=nch, chunked_in=True, b_ref=b_ref)
    else:
        in_specs = [
            pl.BlockSpec((BN, din), lambda i: (i, 0)),
            pl.BlockSpec((din, dout), lambda i: (0, 0)),
            pl.BlockSpec((8, oc), lambda i: (0, 0)),
            pl.BlockSpec((8, oc), lambda i: (0, 0)),
        ]
        args = (x, w, att_s, att_d)

        def body(x_ref, w_ref, as_ref, ad_ref, h_ref, asd_ref, sv_ref):
            _mm_body(x_ref, w_ref, as_ref, ad_ref, h_ref, asd_ref, sv_ref,
                     heads=heads, oc=oc, nch=nch, chunked_in=False)

    cw = dout // nch
    h_resh, asd, sv = pl.pallas_call(
        body,
        grid=grid,
        in_specs=in_specs,
        out_specs=(
            pl.BlockSpec((nch, BN, cw), lambda i: (0, i, 0)),
            pl.BlockSpec((1, 16, BN), lambda i: (i, 0, 0)),
            pl.BlockSpec((8, 128), lambda i: (0, 0)),
        ),
        out_shape=(
            jax.ShapeDtypeStruct((nch, N, cw), jnp.float32),
            jax.ShapeDtypeStruct((N // BN, 16, BN), jnp.float32),
            jax.ShapeDtypeStruct((8, 128), jnp.float32),
        ),
    )(*args)
    asd = asd.transpose(1, 0, 2).reshape(16, N)
    return h_resh, asd, sv


def _pass_a_call(srcp, dstp, asdf, svf, heads):
    epw = EP // NW
    nb = epw // BA
    mesh = plsc.VectorSubcoreMesh(**_SC_MESH)

    @functools.partial(
        pl.kernel,
        out_type=(
            jax.ShapeDtypeStruct((heads * EP,), jnp.float32),
            jax.ShapeDtypeStruct((NW * heads * N2,), jnp.float32),
        ),
        mesh=mesh,
        compiler_params=pltpu.CompilerParams(needs_layout_passes=False),
        scratch_types=[
            pltpu.VMEM((N2,), jnp.float32),
            pltpu.VMEM((N2,), jnp.float32),
            pltpu.VMEM((N2,), jnp.float32),
            pltpu.VMEM((BA,), jnp.int32),
            pltpu.VMEM((BA,), jnp.int32),
            pltpu.VMEM((BA,), jnp.float32),
            pltpu.VMEM((16,), jnp.float32),
            pltpu.VMEM((16,), jnp.float32),
        ],
    )
    def k(src_hbm, dst_hbm, asd_hbm, sv_hbm, ex_hbm, part_hbm,
          s_tbl, d_tbl, denom, srcb, dstb, exb, svs, svd):
        wid = lax.axis_index("s") * NC + lax.axis_index("c")
        base = wid * epw
        for hd in range(heads):
            pltpu.sync_copy(asd_hbm.at[pl.ds(hd * N2, N2)], s_tbl)
            pltpu.sync_copy(asd_hbm.at[pl.ds((8 + hd) * N2, N2)], d_tbl)
            pltpu.sync_copy(sv_hbm.at[pl.ds(hd * 128, 16)], svs)
            pltpu.sync_copy(sv_hbm.at[pl.ds((4 + hd) * 128, 16)], svd)
            shift = svs[pl.ds(0, 16)][0] + svd[pl.ds(0, 16)][0]

            def zb(i, _):
                denom[pl.ds(i * 16, 16)] = jnp.zeros((16,), jnp.float32)
                return 0
            lax.fori_loop(0, N2 // 16, zb, 0)

            def batch(bi, _):
                off = base + bi * BA
                pltpu.sync_copy(src_hbm.at[pl.ds(off, BA)], srcb)
                pltpu.sync_copy(dst_hbm.at[pl.ds(off, BA)], dstb)

                def inner(j, _):
                    sl = pl.ds(j * 16, 16)
                    sv_idx = srcb[sl]
                    dv_idx = dstb[sl]
                    av = plsc.load_gather(s_tbl, [sv_idx])
                    bv = plsc.load_gather(d_tbl, [dv_idx])
                    al = av + bv
                    al = jnp.where(al > 0, al, NEG_SLOPE * al)
                    ex = jnp.exp(al - shift)
                    gid = off + j * 16 + lax.iota(jnp.int32, 16)
                    ex = jnp.where(gid < E_REAL, ex, 0.0)
                    exb[sl] = ex
                    plsc.addupdate_scatter(denom, [dv_idx], ex)
                    return 0
                lax.fori_loop(0, BA // 16, inner, 0)
                pltpu.sync_copy(exb, ex_hbm.at[pl.ds(hd * EP + off, BA)])
                return 0
            lax.fori_loop(0, nb, batch, 0)
            pltpu.sync_copy(
                denom, part_hbm.at[pl.ds(wid * heads * N2 + hd * N2, N2)])

    return k(srcp, dstp, asdf, svf)


def _reduce_call(partials, heads):
    m = heads * N2
    p2 = partials.reshape(NW, m)

    def body(p_ref, o_ref):
        s = jnp.sum(p_ref[...], axis=0)
        o_ref[...] = jnp.broadcast_to((1.0 / s)[None, :], (8, m))

    invd = pl.pallas_call(
        body,
        in_specs=[pl.BlockSpec((NW, m), lambda: (0, 0))],
        out_specs=pl.BlockSpec((8, m), lambda: (0, 0)),
        out_shape=jax.ShapeDtypeStruct((8, m), jnp.float32),
    )(p2)
    return invd.reshape(-1)


def _pass_c_call(srcp, dstp, exf, invdf, tbl, *, heads, ncht, cw):
    nch = ncht // NC
    etw = EP // NS
    nb = etw // BC
    zr = 16
    mesh = plsc.VectorSubcoreMesh(**_SC_MESH)

    @functools.partial(
        pl.kernel,
        out_type=jax.ShapeDtypeStruct((ncht * N, cw), jnp.float32),
        mesh=mesh,
        compiler_params=pltpu.CompilerParams(needs_layout_passes=False),
        scratch_types=[
            pltpu.VMEM_SHARED((N2, cw), jnp.float32),
            pltpu.VMEM((N2,), jnp.float32),
            [pltpu.VMEM((BC,), jnp.int32)] * 2,
            [pltpu.VMEM((BC,), jnp.int32)] * 2,
            [pltpu.VMEM((BC,), jnp.float32)] * 2,
            [pltpu.VMEM((BC,), jnp.float32)] * 2,
            [pltpu.VMEM((BC, cw), jnp.float32)] * 2,
            pltpu.VMEM((zr, cw), jnp.float32),
            [pltpu.SemaphoreType.DMA] * 2,
        ],
    )
    def k(src_hbm, dst_hbm, ex_hbm, invd_hbm, tbl_hbm, out_hbm,
          acc, invd_tbl, srcb, dstb, exb, cfb, rowsb, zbuf, sem):
        cid = lax.axis_index("c")
        sid = lax.axis_index("s")
        ebase = sid * etw

        def zz(r, _):
            for c2 in range(cw // 16):
                zbuf[r, pl.ds(c2 * 16, 16)] = jnp.zeros((16,), jnp.float32)
            return 0
        lax.fori_loop(0, zr, zz, 0)

        def load_adj(bi, b, q, hd):
            eoff = ebase + bi * BC
            pltpu.sync_copy(src_hbm.at[pl.ds(eoff, BC)], srcb[b])
            pltpu.sync_copy(dst_hbm.at[pl.ds(eoff, BC)], dstb[b])
            pltpu.sync_copy(ex_hbm.at[pl.ds(hd * EP + eoff, BC)], exb[b])

            def adj(j, _):
                sl = pl.ds(j * 16, 16)
                srcb[b][sl] = srcb[b][sl] + q * N
                inv = plsc.load_gather(invd_tbl, [dstb[b][sl]])
                cfb[b][sl] = exb[b][sl] * inv
                return 0
            lax.fori_loop(0, BC // 16, adj, 0)
            pltpu.async_copy(tbl_hbm.at[srcb[b]], rowsb[b], sem[b])

        for kk in range(nch):
            q = cid * nch + kk
            hd = q // (ncht // heads) if heads > 1 else q * 0
            for z in range(640 // zr):
                pltpu.sync_copy(
                    zbuf, acc.at[pl.ds(sid * 640 + z * zr, zr)])
            plsc.subcore_barrier()
            pltpu.sync_copy(
                invd_hbm.at[pl.ds(hd * N2, N2)], invd_tbl)

            for b in range(2):
                load_adj(b, b, q, hd)

            def pair(p, _):
                for b in range(2):
                    bi = p * 2 + b
                    pltpu.make_async_copy(
                        tbl_hbm.at[srcb[b]], rowsb[b], sem[b]).wait()

                    def wloop(g, _):
                        cf16 = cfb[b][pl.ds(g * 16, 16)]
                        for l in range(16):
                            cf = cf16[l]
                            r = g * 16 + l
                            for c2 in range(cw // 16):
                                sl2 = pl.ds(c2 * 16, 16)
                                rowsb[b][r, sl2] = rowsb[b][r, sl2] * cf
                        return 0
                    lax.fori_loop(0, BC // 16, wloop, 0)

                    pltpu.sync_copy(rowsb[b], acc.at[dstb[b]], add=True)

                    @pl.when(bi + 2 < nb)
                    def _():
                        load_adj(bi + 2, b, q, hd)
                return 0
            lax.fori_loop(0, nb // 2, pair, 0)
            plsc.subcore_barrier()

            @pl.when(sid < 15)
            def _():
                pltpu.sync_copy(
                    acc.at[pl.ds(sid * 640, 640)],
                    out_hbm.at[pl.ds(q * N + sid * 640, 640)])

            @pl.when(sid == 15)
            def _():
                pltpu.sync_copy(
                    acc.at[pl.ds(9600, 400)],
                    out_hbm.at[pl.ds(q * N + 9600, 400)])

    return k(srcp, dstp, exf, invdf, tbl)


def _tail_call(o3, b3):
    o3r = o3
    b3r = jnp.concatenate(
        [b3.reshape(1, 128), jnp.zeros((7, 128), jnp.float32)], axis=0)

    def body(o_ref, b_ref, emb_ref, pred_ref):
        h = o_ref[...] + b_ref[0][None, :]
        emb_ref[...] = h
        m = jnp.max(h, axis=1, keepdims=True)
        lse = jnp.log(jnp.sum(jnp.exp(h - m), axis=1, keepdims=True)) + m
        pred_ref[...] = h - lse

    emb, pred = pl.pallas_call(
        body,
        grid=(N // BN,),
        in_specs=[
            pl.BlockSpec((BN, 128), lambda i: (i, 0)),
            pl.BlockSpec((8, 128), lambda i: (0, 0)),
        ],
        out_specs=(
            pl.BlockSpec((BN, 128), lambda i: (i, 0)),
            pl.BlockSpec((BN, 128), lambda i: (i, 0)),
        ),
        out_shape=(
            jax.ShapeDtypeStruct((N, 128), jnp.float32),
            jax.ShapeDtypeStruct((N, 128), jnp.float32),
        ),
    )(o3r, b3r)
    return emb, pred


def _pad_att(a, oc):
    a2 = a.reshape(-1, oc)
    return jnp.concatenate(
        [a2, jnp.zeros((8 - a2.shape[0], oc), jnp.float32)], axis=0)


def _gat_layer(x, srcp, dstp, w, att_s, att_d, *, heads, oc, nch,
               agg_ncht, chunked_in, bias=None):
    h_resh, asd, sv = _mm_call(
        x, w, _pad_att(att_s, oc), _pad_att(att_d, oc),
        heads=heads, oc=oc, nch=nch, chunked_in=chunked_in, bias=bias)
    asdf = jnp.concatenate(
        [asd, jnp.zeros((16, N2 - N), jnp.float32)], axis=1).reshape(-1)
    exf, partials = _pass_a_call(srcp, dstp, asdf, sv.reshape(-1), heads)
    invdf = _reduce_call(partials, heads)
    tbl = h_resh.reshape(nch * N, 128)
    if agg_ncht != nch:
        tbl = jnp.concatenate([tbl] * (agg_ncht // nch), axis=0)
    out = _pass_c_call(
        srcp, dstp, exf, invdf, tbl, heads=heads, ncht=agg_ncht, cw=128)
    return out


def kernel(x, edge_index, W1, a_src1, a_dst1, b1, W2, a_src2, a_dst2, b2,
           W3, a_src3, a_dst3, b3):
    loop = jnp.arange(N, dtype=jnp.int32)
    pad = jnp.zeros((EP - E_REAL,), jnp.int32)
    srcp = jnp.concatenate([edge_index[0].astype(jnp.int32), loop, pad])
    dstp = jnp.concatenate([edge_index[1].astype(jnp.int32), loop, pad])

    o1 = _gat_layer(x, srcp, dstp, W1, a_src1, a_dst1,
                    heads=4, oc=256, nch=8, agg_ncht=8, chunked_in=False)
    o2 = _gat_layer(o1.reshape(8, N, 128), srcp, dstp, W2, a_src2, a_dst2,
                    heads=4, oc=256, nch=8, agg_ncht=8, chunked_in=True,
                    bias=b1.reshape(8, 128))
    o3 = _gat_layer(o2.reshape(8, N, 128), srcp, dstp, W3, a_src3, a_dst3,
                    heads=1, oc=128, nch=1, agg_ncht=2, chunked_in=True,
                    bias=b2.reshape(8, 128))
    return _tail_call(o3.reshape(2, N, 128)[0], b3)


# speedup vs baseline: 15.5463x; 1.0529x over previous
import functools

import jax
import jax.numpy as jnp
from jax import lax
from jax.experimental import pallas as pl
from jax.experimental.pallas import tpu as pltpu
from jax.experimental.pallas import tpu_sc as plsc

N = 10000
E_RAW = 160000
E_REAL = E_RAW + N
NC = 2
NS = 16
NW = NC * NS
EP = 172032
N2 = 10240
BA = 384
BC = 128
BN = 1000
NEG_SLOPE = 0.2

_SC_MESH = dict(core_axis_name="c", subcore_axis_name="s", num_cores=NC,
                num_subcores=NS)


def _mm_body(x_ref, w_ref, as_ref, ad_ref, h_ref, asd_ref, sv_ref, *,
             heads, oc, nch, chunked_in, b_ref=None):
    i = pl.program_id(0)
    if chunked_in:
        xs = [jnp.maximum(x_ref[c] + b_ref[c][None, :], 0.0) for c in range(8)]
        x = jnp.concatenate(xs, axis=1)
    else:
        x = x_ref[...]
    h = jnp.dot(x, w_ref[...], preferred_element_type=jnp.float32,
                precision=lax.Precision.HIGHEST)
    cw = h.shape[1] // nch
    for c in range(nch):
        h_ref[c] = h[:, c * cw:(c + 1) * cw]

    rows = []
    maxs = []
    for hd in range(heads):
        hh = h[:, hd * oc:(hd + 1) * oc]
        a_s = jnp.dot(hh, as_ref[hd], preferred_element_type=jnp.float32,
                      precision=lax.Precision.HIGHEST)
        rows.append(a_s)
        maxs.append(jnp.max(a_s))
    for hd in range(heads, 8):
        rows.append(jnp.zeros((BN,), jnp.float32))
        maxs.append(jnp.float32(0.0))
    for hd in range(heads):
        hh = h[:, hd * oc:(hd + 1) * oc]
        a_d = jnp.dot(hh, ad_ref[hd], preferred_element_type=jnp.float32,
                      precision=lax.Precision.HIGHEST)
        rows.append(a_d)
        maxs.append(jnp.max(a_d))
    for hd in range(heads, 8):
        rows.append(jnp.zeros((BN,), jnp.float32))
        maxs.append(jnp.float32(0.0))
    asd_ref[...] = jnp.stack(rows, axis=0)[None]

    mv = jnp.stack([jnp.full((128,), m, jnp.float32) for m in maxs[:8]],
                   axis=0)
    mv2 = jnp.stack([jnp.full((128,), m, jnp.float32) for m in maxs[8:]],
                    axis=0)
    cur = jnp.concatenate([mv[:4], mv2[:4]], axis=0)

    @pl.when(i == 0)
    def _():
        sv_ref[...] = jnp.full((8, 128), -1e30, jnp.float32)

    sv_ref[...] = jnp.maximum(sv_ref[...], cur)


def _mm_call(x, w, att_s, att_d, *, heads, oc, nch, chunked_in, bias=None):
    din = w.shape[0]
    dout = w.shape[1]
    grid = (N // BN,)
    if chunked_in:
        in_specs = [
            pl.BlockSpec((8, BN, 128), lambda i: (0, i, 0)),
            pl.BlockSpec((8, 128), lambda i: (0, 0)),
            pl.BlockSpec((din, dout), lambda i: (0, 0)),
            pl.BlockSpec((8, oc), lambda i: (0, 0)),
            pl.BlockSpec((8, oc), lambda i: (0, 0)),
        ]
        args = (x, bias, w, att_s, att_d)

        def body(x_ref, b_ref, w_ref, as_ref, ad_ref, h_ref, asd_ref, sv_ref):
            _mm_body(x_ref, w_ref, as_ref, ad_ref, h_ref, asd_ref, sv_ref,
                     heads=heads, oc=oc, nch=nch, chunked_in=True, b_ref=b_ref)
    else:
        in_specs = [
            pl.BlockSpec((BN, din), lambda i: (i, 0)),
            pl.BlockSpec((din, dout), lambda i: (0, 0)),
            pl.BlockSpec((8, oc), lambda i: (0, 0)),
            pl.BlockSpec((8, oc), lambda i: (0, 0)),
        ]
        args = (x, w, att_s, att_d)

        def body(x_ref, w_ref, as_ref, ad_ref, h_ref, asd_ref, sv_ref):
            _mm_body(x_ref, w_ref, as_ref, ad_ref, h_ref, asd_ref, sv_ref,
                     heads=heads, oc=oc, nch=nch, chunked_in=False)

    cw = dout // nch
    h_resh, asd, sv = pl.pallas_call(
        body,
        grid=grid,
        in_specs=in_specs,
        out_specs=(
            pl.BlockSpec((nch, BN, cw), lambda i: (0, i, 0)),
            pl.BlockSpec((1, 16, BN), lambda i: (i, 0, 0)),
            pl.BlockSpec((8, 128), lambda i: (0, 0)),
        ),
        out_shape=(
            jax.ShapeDtypeStruct((nch, N, cw), jnp.float32),
            jax.ShapeDtypeStruct((N // BN, 16, BN), jnp.float32),
            jax.ShapeDtypeStruct((8, 128), jnp.float32),
        ),
    )(*args)
    asd = asd.transpose(1, 0, 2).reshape(16, N)
    return h_resh, asd, sv


def _pass_a_call(srcp, dstp, asdf, svf, heads):
    epw = EP // NW
    nb = epw // BA
    mesh = plsc.VectorSubcoreMesh(**_SC_MESH)

    @functools.partial(
        pl.kernel,
        out_type=(
            jax.ShapeDtypeStruct((heads * EP,), jnp.float32),
            jax.ShapeDtypeStruct((NW * heads * N2,), jnp.float32),
        ),
        mesh=mesh,
        compiler_params=pltpu.CompilerParams(needs_layout_passes=False),
        scratch_types=[
            pltpu.VMEM((N2,), jnp.float32),
            pltpu.VMEM((N2,), jnp.float32),
            pltpu.VMEM((N2,), jnp.float32),
            pltpu.VMEM((BA,), jnp.int32),
            pltpu.VMEM((BA,), jnp.int32),
            pltpu.VMEM((BA,), jnp.float32),
            pltpu.VMEM((16,), jnp.float32),
            pltpu.VMEM((16,), jnp.float32),
        ],
    )
    def k(src_hbm, dst_hbm, asd_hbm, sv_hbm, ex_hbm, part_hbm,
          s_tbl, d_tbl, denom, srcb, dstb, exb, svs, svd):
        wid = lax.axis_index("s") * NC + lax.axis_index("c")
        base = wid * epw
        for hd in range(heads):
            pltpu.sync_copy(asd_hbm.at[pl.ds(hd * N2, N2)], s_tbl)
            pltpu.sync_copy(asd_hbm.at[pl.ds((8 + hd) * N2, N2)], d_tbl)
            pltpu.sync_copy(sv_hbm.at[pl.ds(hd * 128, 16)], svs)
            pltpu.sync_copy(sv_hbm.at[pl.ds((4 + hd) * 128, 16)], svd)
            shift = svs[pl.ds(0, 16)][0] + svd[pl.ds(0, 16)][0]

            def zb(i, _):
                denom[pl.ds(i * 16, 16)] = jnp.zeros((16,), jnp.float32)
                return 0
            lax.fori_loop(0, N2 // 16, zb, 0)

            def batch(bi, _):
                off = base + bi * BA
                pltpu.sync_copy(src_hbm.at[pl.ds(off, BA)], srcb)
                pltpu.sync_copy(dst_hbm.at[pl.ds(off, BA)], dstb)

                def inner(j, _):
                    sl = pl.ds(j * 16, 16)
                    sv_idx = srcb[sl]
                    dv_idx = dstb[sl]
                    av = plsc.load_gather(s_tbl, [sv_idx])
                    bv = plsc.load_gather(d_tbl, [dv_idx])
                    al = av + bv
                    al = jnp.where(al > 0, al, NEG_SLOPE * al)
                    ex = jnp.exp(al - shift)
                    gid = off + j * 16 + lax.iota(jnp.int32, 16)
                    ex = jnp.where(gid < E_REAL, ex, 0.0)
                    exb[sl] = ex
                    plsc.addupdate_scatter(denom, [dv_idx], ex)
                    return 0
                lax.fori_loop(0, BA // 16, inner, 0)
                pltpu.sync_copy(exb, ex_hbm.at[pl.ds(hd * EP + off, BA)])
                return 0
            lax.fori_loop(0, nb, batch, 0)
            pltpu.sync_copy(
                denom, part_hbm.at[pl.ds(wid * heads * N2 + hd * N2, N2)])

    return k(srcp, dstp, asdf, svf)


def _reduce_call(partials, heads):
    m = heads * N2
    p2 = partials.reshape(NW, m)

    def body(p_ref, o_ref):
        s = jnp.sum(p_ref[...], axis=0)
        o_ref[...] = jnp.broadcast_to((1.0 / s)[None, :], (8, m))

    invd = pl.pallas_call(
        body,
        in_specs=[pl.BlockSpec((NW, m), lambda: (0, 0))],
        out_specs=pl.BlockSpec((8, m), lambda: (0, 0)),
        out_shape=jax.ShapeDtypeStruct((8, m), jnp.float32),
    )(p2)
    return invd.reshape(-1)


def _pass_c_call(srcp, dstp, exf, invdf, tbl, *, heads, ncht, cw,
                 edge_split=False):
    nch = ncht // NC
    etw = EP // NS // (2 if edge_split else 1)
    nb = etw // BC
    zr = 16
    mesh = plsc.VectorSubcoreMesh(**_SC_MESH)

    @functools.partial(
        pl.kernel,
        out_type=jax.ShapeDtypeStruct((ncht * N, cw), jnp.float32),
        mesh=mesh,
        compiler_params=pltpu.CompilerParams(needs_layout_passes=False),
        scratch_types=[
            pltpu.VMEM_SHARED((N2, cw), jnp.float32),
            pltpu.VMEM((N2,), jnp.float32),
            [pltpu.VMEM((BC,), jnp.int32)] * 2,
            [pltpu.VMEM((BC,), jnp.int32)] * 2,
            [pltpu.VMEM((BC,), jnp.float32)] * 2,
            [pltpu.VMEM((BC,), jnp.float32)] * 2,
            [pltpu.VMEM((BC, cw), jnp.float32)] * 2,
            pltpu.VMEM((zr, cw), jnp.float32),
            [pltpu.SemaphoreType.DMA] * 2,
        ],
    )
    def k(src_hbm, dst_hbm, ex_hbm, invd_hbm, tbl_hbm, out_hbm,
          acc, invd_tbl, srcb, dstb, exb, cfb, rowsb, zbuf, sem):
        cid = lax.axis_index("c")
        sid = lax.axis_index("s")
        if edge_split:
            ebase = cid * (EP // 2) + sid * etw
        else:
            ebase = sid * etw

        def zz(r, _):
            for c2 in range(cw // 16):
                zbuf[r, pl.ds(c2 * 16, 16)] = jnp.zeros((16,), jnp.float32)
            return 0
        lax.fori_loop(0, zr, zz, 0)

        def load_adj(bi, b, q, hd):
            eoff = ebase + bi * BC
            pltpu.sync_copy(src_hbm.at[pl.ds(eoff, BC)], srcb[b])
            pltpu.sync_copy(dst_hbm.at[pl.ds(eoff, BC)], dstb[b])
            pltpu.sync_copy(ex_hbm.at[pl.ds(hd * EP + eoff, BC)], exb[b])

            def adj(j, _):
                sl = pl.ds(j * 16, 16)
                srcb[b][sl] = srcb[b][sl] + q * N
                inv = plsc.load_gather(invd_tbl, [dstb[b][sl]])
                cfb[b][sl] = exb[b][sl] * inv
                return 0
            lax.fori_loop(0, BC // 16, adj, 0)
            pltpu.async_copy(tbl_hbm.at[srcb[b]], rowsb[b], sem[b])

        for kk in range(nch):
            q = cid * nch + kk
            hd = q // (ncht // heads) if heads > 1 else q * 0
            for z in range(640 // zr):
                pltpu.sync_copy(
                    zbuf, acc.at[pl.ds(sid * 640 + z * zr, zr)])
            plsc.subcore_barrier()
            pltpu.sync_copy(
                invd_hbm.at[pl.ds(hd * N2, N2)], invd_tbl)

            for b in range(2):
                load_adj(b, b, q, hd)

            def pair(p, _):
                for b in range(2):
                    bi = p * 2 + b
                    pltpu.make_async_copy(
                        tbl_hbm.at[srcb[b]], rowsb[b], sem[b]).wait()

                    def wloop(g, _):
                        cf16 = cfb[b][pl.ds(g * 16, 16)]
                        for l in range(16):
                            cf = cf16[l]
                            r = g * 16 + l
                            for c2 in range(cw // 16):
                                sl2 = pl.ds(c2 * 16, 16)
                                rowsb[b][r, sl2] = rowsb[b][r, sl2] * cf
                        return 0
                    lax.fori_loop(0, BC // 16, wloop, 0)

                    pltpu.sync_copy(rowsb[b], acc.at[dstb[b]], add=True)

                    @pl.when(bi + 2 < nb)
                    def _():
                        load_adj(bi + 2, b, q, hd)
                return 0
            lax.fori_loop(0, nb // 2, pair, 0)
            plsc.subcore_barrier()

            @pl.when(sid < 15)
            def _():
                pltpu.sync_copy(
                    acc.at[pl.ds(sid * 640, 640)],
                    out_hbm.at[pl.ds(q * N + sid * 640, 640)])

            @pl.when(sid == 15)
            def _():
                pltpu.sync_copy(
                    acc.at[pl.ds(9600, 400)],
                    out_hbm.at[pl.ds(q * N + 9600, 400)])

    return k(srcp, dstp, exf, invdf, tbl)


def _tail_call(o3, b3):
    o3r = o3.reshape(2, N, 128)
    b3r = jnp.concatenate(
        [b3.reshape(1, 128), jnp.zeros((7, 128), jnp.float32)], axis=0)

    def body(o_ref, b_ref, emb_ref, pred_ref):
        h = o_ref[0] + o_ref[1] + b_ref[0][None, :]
        emb_ref[...] = h
        m = jnp.max(h, axis=1, keepdims=True)
        lse = jnp.log(jnp.sum(jnp.exp(h - m), axis=1, keepdims=True)) + m
        pred_ref[...] = h - lse

    emb, pred = pl.pallas_call(
        body,
        grid=(N // BN,),
        in_specs=[
            pl.BlockSpec((2, BN, 128), lambda i: (0, i, 0)),
            pl.BlockSpec((8, 128), lambda i: (0, 0)),
        ],
        out_specs=(
            pl.BlockSpec((BN, 128), lambda i: (i, 0)),
            pl.BlockSpec((BN, 128), lambda i: (i, 0)),
        ),
        out_shape=(
            jax.ShapeDtypeStruct((N, 128), jnp.float32),
            jax.ShapeDtypeStruct((N, 128), jnp.float32),
        ),
    )(o3r, b3r)
    return emb, pred


def _pad_att(a, oc):
    a2 = a.reshape(-1, oc)
    return jnp.concatenate(
        [a2, jnp.zeros((8 - a2.shape[0], oc), jnp.float32)], axis=0)


def _gat_layer(x, srcp, dstp, w, att_s, att_d, *, heads, oc, nch,
               agg_ncht, chunked_in, bias=None):
    h_resh, asd, sv = _mm_call(
        x, w, _pad_att(att_s, oc), _pad_att(att_d, oc),
        heads=heads, oc=oc, nch=nch, chunked_in=chunked_in, bias=bias)
    asdf = jnp.concatenate(
        [asd, jnp.zeros((16, N2 - N), jnp.float32)], axis=1).reshape(-1)
    exf, partials = _pass_a_call(srcp, dstp, asdf, sv.reshape(-1), heads)
    invdf = _reduce_call(partials, heads)
    tbl = h_resh.reshape(nch * N, 128)
    if agg_ncht != nch:
        tbl = jnp.concatenate([tbl] * (agg_ncht // nch), axis=0)
    out = _pass_c_call(
        srcp, dstp, exf, invdf, tbl, heads=heads, ncht=agg_ncht, cw=128,
        edge_split=(agg_ncht != nch))
    return out


def kernel(x, edge_index, W1, a_src1, a_dst1, b1, W2, a_src2, a_dst2, b2,
           W3, a_src3, a_dst3, b3):
    loop = jnp.arange(N, dtype=jnp.int32)
    pad = jnp.zeros((EP - E_REAL,), jnp.int32)
    srcp = jnp.concatenate([edge_index[0].astype(jnp.int32), loop, pad])
    dstp = jnp.concatenate([edge_index[1].astype(jnp.int32), loop, pad])

    o1 = _gat_layer(x, srcp, dstp, W1, a_src1, a_dst1,
                    heads=4, oc=256, nch=8, agg_ncht=8, chunked_in=False)
    o2 = _gat_layer(o1.reshape(8, N, 128), srcp, dstp, W2, a_src2, a_dst2,
                    heads=4, oc=256, nch=8, agg_ncht=8, chunked_in=True,
                    bias=b1.reshape(8, 128))
    o3 = _gat_layer(o2.reshape(8, N, 128), srcp, dstp, W3, a_src3, a_dst3,
                    heads=1, oc=128, nch=1, agg_ncht=2, chunked_in=True,
                    bias=b2.reshape(8, 128))
    return _tail_call(o3, b3)
